# Initial kernel scaffold; baseline (speedup 1.0000x reference)
import jax
import jax.numpy as jnp
from jax.experimental import pallas as pl


def kernel(x_zone, x_equipment, x_surface, ei_zz, ei_ze, ei_ez, ei_sz, ei_zs, building_type_idx, W_emb_zone, b_emb_zone, W_emb_equipment, b_emb_equipment, W_emb_surface, b_emb_surface, W1_zz, b1_zz, W1_ze, b1_ze, W1_ez, b1_ez, W1_sz, b1_sz, W1_zs, b1_zs, W2_zz, b2_zz, W2_ze, b2_ze, W2_ez, b2_ez, W2_sz, b2_sz, W2_zs, b2_zs, bt_table, W_fc1, b_fc1, W_fc2, b_fc2):
    raise NotImplementedError("write your pallas kernel here")


# speedup vs baseline: 15.4194x; 15.4194x over previous
import jax
import jax.numpy as jnp
from jax import lax
from jax.experimental import pallas as pl
from jax.experimental.pallas import tpu as pltpu
from jax.experimental.pallas import tpu_sc as plsc

N = 50000
NP = 51200
E = 800000
EP = 819200
ROWS_E = EP // 128
TPT = ROWS_E // 32
RPT = NP // 16
HH = 32
BLK = 1000
GRID = N // BLK
NCORE, NSUB = 2, 16

_R = ('zz', 'ze', 'ez', 'sz', 'zs')
_SRC = {'zz': 0, 'ze': 0, 'ez': 1, 'sz': 2, 'zs': 0}

_mesh = plsc.VectorSubcoreMesh(core_axis_name="c", subcore_axis_name="s",
                               num_cores=NCORE, num_subcores=NSUB)
_sc_params = pltpu.CompilerParams(use_tc_tiling_on_sc=False)


def _degrees_body(*refs):
    ins = refs[0:10]
    z1d = refs[10]
    outs = refs[11:31]
    accs = refs[31:41]
    idxb, ones, sem = refs[41], refs[42], refs[43]
    cid = lax.axis_index("c")
    sid = lax.axis_index("s")
    wid = sid * NCORE + cid

    def fill_ones(i, _):
        ones[pl.ds(i * 16, 16)] = jnp.ones((16,), jnp.float32)
        return 0
    lax.fori_loop(0, 8, fill_ones, 0)

    base = sid * RPT
    for h in range(10):
        pltpu.sync_copy(z1d, accs[h].at[pl.ds(base, RPT)])
    plsc.subcore_barrier()

    erow0 = wid * TPT
    for h in range(10):
        pltpu.sync_copy(ins[h].at[pl.ds(erow0, TPT)], idxb)
        for k in range(8):
            pltpu.async_copy(ones, accs[h].at[idxb.at[k]], sem, add=True)

        def grp(g, _):
            @pl.when(g < 24)
            def _fire():
                for k in range(8):
                    pltpu.async_copy(ones, accs[h].at[idxb.at[(g + 1) * 8 + k]],
                                     sem, add=True)
            for k in range(8):
                pltpu.make_async_copy(ones, accs[h].at[idxb.at[g * 8 + k]],
                                      sem).wait()
            return 0
        lax.fori_loop(0, 25, grp, 0)
    plsc.subcore_barrier()

    for c in range(2):
        @pl.when(cid == c)
        def _wb():
            for h in range(10):
                pltpu.sync_copy(accs[h].at[pl.ds(base, RPT)],
                                outs[h * 2 + c].at[pl.ds(base, RPT)])


_degrees = pl.kernel(
    _degrees_body,
    out_type=tuple(jax.ShapeDtypeStruct((NP,), jnp.float32) for _ in range(20)),
    mesh=_mesh,
    scratch_types=(
        [pltpu.VMEM_SHARED((NP,), jnp.float32) for _ in range(10)]
        + [pltpu.VMEM((TPT, 128), jnp.int32),
           pltpu.VMEM((128,), jnp.float32),
           pltpu.SemaphoreType.DMA]),
    compiler_params=_sc_params,
)


def _msgpass_body(*refs):
    srcg = refs[0:5]
    dstp = refs[5:10]
    srch = refs[10:15]
    T = refs[15:25]
    nd = refs[25:30]
    z2d = refs[30]
    z1d = refs[31]
    A = refs[32:52]
    C = refs[52:62]
    accv, cacc, sbuf, dbuf, rows, vals, sem = refs[62:69]
    cid = lax.axis_index("c")
    sid = lax.axis_index("s")
    wid = sid * NCORE + cid

    base = sid * RPT
    erow0 = wid * TPT
    SEG = 40
    NSEG = TPT // SEG
    for ri in range(5):
        for half in range(2):
            tab = T[ri * 2 + half]

            pltpu.sync_copy(z2d, accv.at[pl.ds(base, RPT)])
            plsc.subcore_barrier()

            def seg_body(s2, _):
                r0 = erow0 + s2 * SEG
                pltpu.sync_copy(srcg[ri].at[pl.ds(r0, SEG)], sbuf)
                pltpu.sync_copy(dstp[ri].at[pl.ds(r0, SEG)], dbuf)
                for k in range(2):
                    pltpu.async_copy(tab.at[sbuf.at[k]], rows.at[k], sem)

                def mp(j, _):
                    @pl.when(j + 2 < SEG)
                    def _fire():
                        pltpu.async_copy(tab.at[sbuf.at[j + 2]],
                                         rows.at[lax.rem(j + 2, 3)], sem)
                    b = lax.rem(j, 3)
                    pltpu.make_async_copy(tab.at[sbuf.at[j]], rows.at[b],
                                          sem).wait()
                    pltpu.sync_copy(rows.at[b], accv.at[dbuf.at[j]], add=True)
                    return 0
                lax.fori_loop(0, SEG, mp, 0)
                return 0
            lax.fori_loop(0, NSEG, seg_body, 0)
            plsc.subcore_barrier()

            for c in range(2):
                @pl.when(cid == c)
                def _wb():
                    def wb(k, _):
                        pltpu.sync_copy(
                            accv.at[pl.ds(base + k * 800, 800)],
                            A[(ri * 2 + half) * 2 + c].at[pl.ds(base + k * 800, 800)])
                        return 0
                    lax.fori_loop(0, 4, wb, 0)

        pltpu.sync_copy(z1d, cacc.at[pl.ds(base, RPT)])
        plsc.subcore_barrier()

        def cseg(s2, _):
            r0 = erow0 + s2 * SEG
            pltpu.sync_copy(srch[ri].at[pl.ds(r0, SEG)], sbuf)
            pltpu.sync_copy(dstp[ri].at[pl.ds(r0, SEG)], dbuf)
            for k in range(7):
                pltpu.async_copy(nd[ri].at[dbuf.at[k]], vals.at[k], sem)

            def cp(j, _):
                @pl.when(j + 7 < SEG)
                def _fire():
                    pltpu.async_copy(nd[ri].at[dbuf.at[j + 7]],
                                     vals.at[lax.rem(j + 7, 8)], sem)
                b = lax.rem(j, 8)
                pltpu.make_async_copy(nd[ri].at[dbuf.at[j]], vals.at[b],
                                      sem).wait()
                pltpu.sync_copy(vals.at[b], cacc.at[sbuf.at[j]], add=True)
                return 0
            lax.fori_loop(0, SEG, cp, 0)
            return 0
        lax.fori_loop(0, NSEG, cseg, 0)
        plsc.subcore_barrier()
        for c in range(2):
            @pl.when(cid == c)
            def _wbc():
                def wbc(k, _):
                    pltpu.sync_copy(cacc.at[pl.ds(base + k * 800, 800)],
                                    C[ri * 2 + c].at[pl.ds(base + k * 800, 800)])
                    return 0
                lax.fori_loop(0, 4, wbc, 0)


_msgpass = pl.kernel(
    _msgpass_body,
    out_type=tuple([jax.ShapeDtypeStruct((NP, HH), jnp.float32) for _ in range(20)]
                   + [jax.ShapeDtypeStruct((NP,), jnp.float32) for _ in range(10)]),
    mesh=_mesh,
    scratch_types=(
        [pltpu.VMEM_SHARED((NP, HH), jnp.float32),
         pltpu.VMEM_SHARED((NP,), jnp.float32),
         pltpu.VMEM((40, 128), jnp.int32),
         pltpu.VMEM((40, 128), jnp.int32),
         pltpu.VMEM((3, 128, HH), jnp.float32),
         pltpu.VMEM((8, 128), jnp.float32),
         pltpu.SemaphoreType.DMA]),
    compiler_params=_sc_params,
)


def _norm(deg):
    return jnp.where(deg > 0, lax.rsqrt(jnp.maximum(deg, 1.0)), 0.0)


def _tables_body(*refs):
    xz, xe, xs = refs[0:3]
    dsp = refs[3:13]
    wemb = refs[13:16]
    bemb = refs[16:19]
    w1 = refs[19:24]
    outs = refs[24:34]
    xsrc = (xz, xe, xs)
    for ri, r in enumerate(_R):
        s = _SRC[r]
        ns = _norm(dsp[2 * ri][...] + dsp[2 * ri + 1][...])
        wc = jnp.dot(wemb[s][...], w1[ri][...],
                     preferred_element_type=jnp.float32)
        bc = jnp.dot(bemb[s][...], w1[ri][...],
                     preferred_element_type=jnp.float32)
        t = (jnp.dot(xsrc[s][...], wc, preferred_element_type=jnp.float32)
             + bc) * ns
        outs[2 * ri][...] = t[:, :HH]
        outs[2 * ri + 1][...] = t[:, HH:]


def _tables_call(xz, xe, xs, dsp_cols, wembs, bembs, w1s):
    full = lambda shp: pl.BlockSpec(shp, lambda i: (0, 0))
    return pl.pallas_call(
        _tables_body,
        grid=(GRID,),
        in_specs=([pl.BlockSpec((BLK, 128), lambda i: (i, 0))] * 3
                  + [pl.BlockSpec((BLK, 1), lambda i: (i, 0))] * 10
                  + [full((128, 64))] * 3 + [full((1, 64))] * 3
                  + [full((64, 64))] * 5),
        out_specs=[pl.BlockSpec((BLK, HH), lambda i: (i, 0))] * 10,
        out_shape=[jax.ShapeDtypeStruct((N, HH), jnp.float32)] * 10,
    )(xz, xe, xs, *dsp_cols, *wembs, *bembs, *w1s)


def _nd_body(*refs):
    dps = refs[0:10]
    outs = refs[10:15]
    for ri in range(5):
        outs[ri][...] = _norm(dps[2 * ri][...] + dps[2 * ri + 1][...])


def _nd_call(dd_parts):
    spec = pl.BlockSpec((NP // 128, 128), lambda: (0, 0))
    return pl.pallas_call(
        _nd_body,
        grid=(),
        in_specs=[spec] * 10,
        out_specs=[spec] * 5,
        out_shape=[jax.ShapeDtypeStruct((NP // 128, 128), jnp.float32)] * 5,
    )(*dd_parts)


def _combine_body(*refs):
    a = refs[0:20]
    cps = refs[20:30]
    dsp = refs[30:40]
    ddp = refs[40:50]
    b1 = refs[50:55]
    w2 = refs[55:60]
    b2 = refs[60:65]
    bt, wfc1, bfc1, wfc2, bfc2, bti = refs[65:71]
    out = refs[71]
    pacc = refs[72]

    i = pl.program_id(0)

    @pl.when(i == 0)
    def _init():
        pacc[...] = jnp.zeros((8, 64), jnp.float32)

    def afull(ri):
        lo = a[4 * ri][...] + a[4 * ri + 1][...]
        hi = a[4 * ri + 2][...] + a[4 * ri + 3][...]
        return jnp.concatenate([lo, hi], axis=1)

    ndv = [_norm(ddp[2 * ri][...] + ddp[2 * ri + 1][...]) for ri in range(5)]
    nsv = [_norm(dsp[2 * ri][...] + dsp[2 * ri + 1][...]) for ri in range(5)]
    alph = [nsv[ri] * (cps[2 * ri][...] + cps[2 * ri + 1][...]) for ri in range(5)]

    izz, ize, iez, isz, izs = 0, 1, 2, 3, 4
    oz = (ndv[izz] * afull(izz) + b1[izz][...]
          + ndv[iez] * afull(iez) + b1[iez][...]
          + ndv[isz] * afull(isz) + b1[isz][...]) / 3.0
    oe = ndv[ize] * afull(ize) + b1[ize][...]
    osf = ndv[izs] * afull(izs) + b1[izs][...]
    rz = jnp.maximum(oz, 0.0)
    re = jnp.maximum(oe, 0.0)
    rs = jnp.maximum(osf, 0.0)
    srcr = (rz, rz, re, rs, rz)

    upd = jnp.concatenate(
        [jnp.sum(srcr[ri] * alph[ri], axis=0, keepdims=True) for ri in range(5)]
        + [jnp.zeros((3, 64), jnp.float32)], axis=0)
    pacc[...] = pacc[...] + upd

    @pl.when(i == GRID - 1)
    def _fin():
        p = pacc[...]
        dot = lambda u, wr: jnp.dot(u, wr[...], preferred_element_type=jnp.float32)
        mz = ((dot(p[izz:izz + 1, :], w2[izz]) + dot(p[iez:iez + 1, :], w2[iez])
               + dot(p[isz:isz + 1, :], w2[isz])) / N
              + b2[izz][...] + b2[iez][...] + b2[isz][...]) / 3.0
        me = dot(p[ize:ize + 1, :], w2[ize]) / N + b2[ize][...]
        ms = dot(p[izs:izs + 1, :], w2[izs]) / N + b2[izs][...]
        ge = jnp.concatenate([mz, me, ms], axis=1)
        sel = lax.broadcasted_iota(jnp.int32, (10, 64), 0) == bti[0, 0]
        btrow = jnp.sum(jnp.where(sel, bt[...], 0.0), axis=0, keepdims=True)
        comb = jnp.concatenate([ge, btrow], axis=1)
        h = jnp.maximum(jnp.dot(comb, wfc1[...],
                                preferred_element_type=jnp.float32) + bfc1[...], 0.0)
        out[...] = jnp.dot(h, wfc2[...],
                           preferred_element_type=jnp.float32) + bfc2[...]


def _combine_call(aparts, cparts, dsp_cols, ddp_cols, b1s, w2s, b2s,
                  bt, wfc1, bfc1, wfc2, bfc2, bti):
    full = lambda shp: pl.BlockSpec(shp, lambda i: (0, 0))
    col = pl.BlockSpec((BLK, 1), lambda i: (i, 0))
    return pl.pallas_call(
        _combine_body,
        grid=(GRID,),
        in_specs=([pl.BlockSpec((BLK, HH), lambda i: (i, 0))] * 20
                  + [col] * 30
                  + [full((1, 64))] * 5 + [full((64, 64))] * 5 + [full((1, 64))] * 5
                  + [full((10, 64)), full((256, 64)), full((1, 64)),
                     full((64, 1)), full((1, 1)),
                     pl.BlockSpec(memory_space=pltpu.SMEM)]),
        out_specs=pl.BlockSpec((1, 1), lambda i: (0, 0)),
        out_shape=jax.ShapeDtypeStruct((1, 1), jnp.float32),
        scratch_shapes=[pltpu.VMEM((8, 64), jnp.float32)],
    )(*aparts, *cparts, *dsp_cols, *ddp_cols, *b1s, *w2s, *b2s,
      bt, wfc1, bfc1, wfc2, bfc2, bti)


def kernel(x_zone, x_equipment, x_surface, ei_zz, ei_ze, ei_ez, ei_sz, ei_zs,
           building_type_idx, W_emb_zone, b_emb_zone, W_emb_equipment,
           b_emb_equipment, W_emb_surface, b_emb_surface, W1_zz, b1_zz, W1_ze,
           b1_ze, W1_ez, b1_ez, W1_sz, b1_sz, W1_zs, b1_zs, W2_zz, b2_zz, W2_ze,
           b2_ze, W2_ez, b2_ez, W2_sz, b2_sz, W2_zs, b2_zs, bt_table, W_fc1,
           b_fc1, W_fc2, b_fc2):
    ei = {'zz': ei_zz, 'ze': ei_ze, 'ez': ei_ez, 'sz': ei_sz, 'zs': ei_zs}
    npad = EP - E
    pad_trash = jnp.full((npad,), N, jnp.int32)
    pad_zero = jnp.zeros((npad,), jnp.int32)
    srch2d = [jnp.concatenate([ei[r][0], pad_trash]).reshape(ROWS_E, 128)
              for r in _R]
    srcg2d = [jnp.concatenate([ei[r][0], pad_zero]).reshape(ROWS_E, 128)
              for r in _R]
    dstp2d = [jnp.concatenate([ei[r][1], pad_trash]).reshape(ROWS_E, 128)
              for r in _R]

    z1d = jnp.zeros((RPT,), jnp.float32)
    z2d = jnp.zeros((RPT, HH), jnp.float32)
    degs = _degrees(*srch2d, *dstp2d, z1d)
    deg_s = degs[0:10]
    deg_d = degs[10:20]

    dsp_cols = [p[:N].reshape(N, 1) for p in deg_s]
    wembs = (W_emb_zone, W_emb_equipment, W_emb_surface)
    bembs = (b_emb_zone.reshape(1, 64), b_emb_equipment.reshape(1, 64),
             b_emb_surface.reshape(1, 64))
    w1s = (W1_zz, W1_ze, W1_ez, W1_sz, W1_zs)
    tables = _tables_call(x_zone, x_equipment, x_surface, dsp_cols, wembs,
                          bembs, w1s)

    dd_parts = [p.reshape(NP // 128, 128) for p in deg_d]
    nd_mats = _nd_call(dd_parts)
    nd_flat = [m.reshape(NP) for m in nd_mats]

    couts = _msgpass(*srcg2d, *dstp2d, *srch2d, *tables, *nd_flat, z2d, z1d)
    aparts = list(couts[0:20])
    craw = couts[20:30]

    cparts = [p[:N].reshape(N, 1) for p in craw]
    ddp_cols = [p[:N].reshape(N, 1) for p in deg_d]
    b1s = (b1_zz.reshape(1, 64), b1_ze.reshape(1, 64), b1_ez.reshape(1, 64),
           b1_sz.reshape(1, 64), b1_zs.reshape(1, 64))
    w2s = (W2_zz, W2_ze, W2_ez, W2_sz, W2_zs)
    b2s = (b2_zz.reshape(1, 64), b2_ze.reshape(1, 64), b2_ez.reshape(1, 64),
           b2_sz.reshape(1, 64), b2_zs.reshape(1, 64))
    bti = jnp.asarray(building_type_idx, jnp.int32).reshape(1, 1)
    return _combine_call(aparts, cparts, dsp_cols, ddp_cols, b1s, w2s, b2s,
                         bt_table, W_fc1, b_fc1.reshape(1, 64), W_fc2,
                         b_fc2.reshape(1, 1), bti)
